# baseline (speedup 1.0000x reference)
import jax
import jax.numpy as jnp
from jax.experimental import pallas as pl
from jax.experimental.pallas import tpu as pltpu

_BS = 32
_S = 4
_NEG = -1e30


def _silu(x):
    return x * jax.nn.sigmoid(x)


def _fwd(q_ref, k_ref, v_ref, k0_ref, v0_ref, gw_ref, o_ref):
    q = q_ref[0, :, 0, 0, :]
    k = k_ref[0, :, 0, 0, :]
    v = v_ref[0, :, 0, 0, :]
    k0 = k0_ref[0, :, 0, 0, :]
    v0 = v0_ref[0, :, 0, 0, :]
    gw = gw_ref[0]

    N, D = q.shape
    nb = N // _BS
    scale = D ** (-0.5)
    f32 = jnp.float32

    e_row = jax.lax.broadcasted_iota(jnp.int32, (nb, N), 0)
    e_col = jax.lax.broadcasted_iota(jnp.int32, (nb, N), 1)
    ind = (e_col // _BS == e_row).astype(f32)
    mean_mat = ind * (1.0 / _BS)

    kc = jnp.dot(mean_mat, k, preferred_element_type=f32, precision=jax.lax.Precision.HIGHEST)
    kc0 = jnp.dot(mean_mat, k0, preferred_element_type=f32, precision=jax.lax.Precision.HIGHEST)
    vc0 = jnp.dot(mean_mat, v0, preferred_element_type=f32, precision=jax.lax.Precision.HIGHEST)

    gates = jax.nn.sigmoid(jnp.dot(q, gw, preferred_element_type=f32, precision=jax.lax.Precision.HIGHEST))
    g_cmp = gates[:, 0:1]
    g_slc = gates[:, 1:2]

    qb = jax.lax.broadcasted_iota(jnp.int32, (N, nb), 0) // _BS
    jb = jax.lax.broadcasted_iota(jnp.int32, (N, nb), 1)
    blk_causal = qb >= jb

    s_sel = jnp.dot(q.astype(jnp.bfloat16), kc.astype(jnp.bfloat16).T,
                    preferred_element_type=f32) * scale
    s_m = jnp.where(blk_causal, s_sel, _NEG)
    sel = jnp.zeros((N, nb), dtype=jnp.bool_)
    for _ in range(_S):
        smax = jnp.max(s_m, axis=1, keepdims=True)
        is_max = s_m == smax
        first = jnp.min(jnp.where(is_max, jb, nb), axis=1, keepdims=True)
        pick = jb == first
        valid = smax > (_NEG * 0.5)
        sel = jnp.logical_or(sel, jnp.logical_and(pick, valid))
        s_m = jnp.where(pick, _NEG, s_m)
    sel_w = sel.astype(f32)

    bf16 = jnp.bfloat16
    q16 = q.astype(bf16)

    sc0 = jnp.dot(q16, kc0.astype(bf16).T, preferred_element_type=f32) * scale
    p_cmp = jnp.where(blk_causal, _silu(sc0), 0.0)
    o_cmp = jnp.dot(p_cmp.astype(bf16), vc0.astype(bf16),
                    preferred_element_type=f32)

    s_tok = jnp.dot(q16, k.astype(bf16).T, preferred_element_type=f32) * scale
    t_row = jax.lax.broadcasted_iota(jnp.int32, (N, N), 0)
    t_col = jax.lax.broadcasted_iota(jnp.int32, (N, N), 1)
    tok_causal = t_row >= t_col
    sel_tok = jnp.dot(sel_w.astype(bf16), ind.astype(bf16),
                      preferred_element_type=f32)
    p = jnp.where(tok_causal, _silu(s_tok), 0.0) * sel_tok
    o_slc = jnp.dot(p.astype(bf16), v.astype(bf16),
                    preferred_element_type=f32)

    o_ref[0, :, 0, 0, :] = o_cmp * g_cmp + o_slc * g_slc


def kernel(jagged_q, jagged_k, jagged_v, jagged_u, padded_q, padded_k,
           padded_v, x_offsets, gate_w, padding_mask):
    B, N, H, D = padded_q.shape
    q5 = padded_q.reshape(B, N, H, 1, D)
    k5 = padded_k.reshape(B, N, H, 1, D)
    v5 = padded_v.reshape(B, N, H, 1, D)

    bhspec = pl.BlockSpec((1, N, 1, 1, D), lambda b, h: (b, 0, h, 0, 0))
    b0spec = pl.BlockSpec((1, N, 1, 1, D), lambda b, h: (0, 0, h, 0, 0))
    gwspec = pl.BlockSpec((1, D, 3), lambda b, h: (h, 0, 0))

    out = pl.pallas_call(
        _fwd,
        grid=(B, H),
        in_specs=[bhspec, bhspec, bhspec, b0spec, b0spec, gwspec],
        out_specs=bhspec,
        out_shape=jax.ShapeDtypeStruct((B, N, H, 1, D), jnp.float32),
        compiler_params=pltpu.CompilerParams(
            dimension_semantics=("parallel", "parallel")),
    )(q5, k5, v5, k5, v5, gate_w)

    return out.reshape(B * N, H, D)


# speedup vs baseline: 1.0042x; 1.0042x over previous
import jax
import jax.numpy as jnp
from jax.experimental import pallas as pl
from jax.experimental.pallas import tpu as pltpu

_BS = 32
_S = 4
_NEG = -1e30


def _silu(x):
    return x * jax.nn.sigmoid(x)


def _fwd(q_ref, k_ref, v_ref, k0_ref, v0_ref, gw_ref, o_ref):
    q = q_ref[0, 0]
    k = k_ref[0, 0]
    v = v_ref[0, 0]
    k0 = k0_ref[0, 0]
    v0 = v0_ref[0, 0]
    gw = gw_ref[0]

    N, D = q.shape
    nb = N // _BS
    scale = D ** (-0.5)
    f32 = jnp.float32

    e_row = jax.lax.broadcasted_iota(jnp.int32, (nb, N), 0)
    e_col = jax.lax.broadcasted_iota(jnp.int32, (nb, N), 1)
    ind = (e_col // _BS == e_row).astype(f32)
    mean_mat = ind * (1.0 / _BS)

    kc = jnp.dot(mean_mat, k, preferred_element_type=f32, precision=jax.lax.Precision.HIGHEST)
    kc0 = jnp.dot(mean_mat, k0, preferred_element_type=f32, precision=jax.lax.Precision.HIGHEST)
    vc0 = jnp.dot(mean_mat, v0, preferred_element_type=f32, precision=jax.lax.Precision.HIGHEST)

    gates = jax.nn.sigmoid(jnp.dot(q, gw, preferred_element_type=f32, precision=jax.lax.Precision.HIGHEST))
    g_cmp = gates[:, 0:1]
    g_slc = gates[:, 1:2]

    qb = jax.lax.broadcasted_iota(jnp.int32, (N, nb), 0) // _BS
    jb = jax.lax.broadcasted_iota(jnp.int32, (N, nb), 1)
    blk_causal = qb >= jb

    s_sel = jnp.dot(q.astype(jnp.bfloat16), kc.astype(jnp.bfloat16).T,
                    preferred_element_type=f32) * scale
    s_m = jnp.where(blk_causal, s_sel, _NEG)
    sel = jnp.zeros((N, nb), dtype=jnp.bool_)
    for _ in range(_S):
        smax = jnp.max(s_m, axis=1, keepdims=True)
        is_max = s_m == smax
        first = jnp.min(jnp.where(is_max, jb, nb), axis=1, keepdims=True)
        pick = jb == first
        valid = smax > (_NEG * 0.5)
        sel = jnp.logical_or(sel, jnp.logical_and(pick, valid))
        s_m = jnp.where(pick, _NEG, s_m)
    sel_w = sel.astype(f32)

    bf16 = jnp.bfloat16
    q16 = q.astype(bf16)

    sc0 = jnp.dot(q16, kc0.astype(bf16).T, preferred_element_type=f32) * scale
    p_cmp = jnp.where(blk_causal, _silu(sc0), 0.0)
    o_cmp = jnp.dot(p_cmp.astype(bf16), vc0.astype(bf16),
                    preferred_element_type=f32)

    k16 = k.astype(bf16)
    v16 = v.astype(bf16)
    sel16 = sel_w.astype(bf16)
    ind16 = ind.astype(bf16)
    ch = 128
    for i in range(N // ch):
        r0 = i * ch
        ncols = r0 + ch
        qi = q16[r0:ncols]
        s = jnp.dot(qi, k16[:ncols].T, preferred_element_type=f32) * scale
        c_row = jax.lax.broadcasted_iota(jnp.int32, (ch, ncols), 0) + r0
        c_col = jax.lax.broadcasted_iota(jnp.int32, (ch, ncols), 1)
        causal = c_row >= c_col
        selt = jnp.dot(sel16[r0:ncols], ind16[:, :ncols],
                       preferred_element_type=f32)
        p = jnp.where(causal, _silu(s), 0.0) * selt
        o_slc = jnp.dot(p.astype(bf16), v16[:ncols],
                        preferred_element_type=f32)
        o_ref[0, 0, r0:ncols, :] = (o_cmp[r0:ncols] * g_cmp[r0:ncols]
                                    + o_slc * g_slc[r0:ncols])


def kernel(jagged_q, jagged_k, jagged_v, jagged_u, padded_q, padded_k,
           padded_v, x_offsets, gate_w, padding_mask):
    B, N, H, D = padded_q.shape
    qt = padded_q.transpose(0, 2, 1, 3)
    kt = padded_k.transpose(0, 2, 1, 3)
    vt = padded_v.transpose(0, 2, 1, 3)

    bhspec = pl.BlockSpec((1, 1, N, D), lambda b, h: (b, h, 0, 0))
    b0spec = pl.BlockSpec((1, 1, N, D), lambda b, h: (0, h, 0, 0))
    gwspec = pl.BlockSpec((1, D, 3), lambda b, h: (h, 0, 0))

    out = pl.pallas_call(
        _fwd,
        grid=(B, H),
        in_specs=[bhspec, bhspec, bhspec, b0spec, b0spec, gwspec],
        out_specs=bhspec,
        out_shape=jax.ShapeDtypeStruct((B, H, N, D), jnp.float32),
        compiler_params=pltpu.CompilerParams(
            dimension_semantics=("parallel", "parallel")),
    )(qt, kt, vt, kt, vt, gate_w)

    return out.transpose(0, 2, 1, 3).reshape(B * N, H, D)


# speedup vs baseline: 1.7135x; 1.7064x over previous
import jax
import jax.numpy as jnp
from jax.experimental import pallas as pl
from jax.experimental.pallas import tpu as pltpu

_BS = 32
_S = 4
_NEG = -1e30


def _silu(x):
    return x * jax.nn.sigmoid(x)


def _dot_tt(a, b):
    return jax.lax.dot_general(a, b, (((0,), (0,)), ((), ())),
                               preferred_element_type=jnp.float32)


def _dot_nt(a, b):
    return jax.lax.dot_general(a, b, (((1,), (1,)), ((), ())),
                               preferred_element_type=jnp.float32)


def _fwd(q_ref, k_ref, v_ref, k0_ref, v0_ref, gw_ref, causal_ref, o_ref):
    q = q_ref[0, 0]
    k = k_ref[0, 0]
    v = v_ref[0, 0]
    k0 = k0_ref[0, 0]
    v0 = v0_ref[0, 0]
    gw = gw_ref[0]
    causal_t = causal_ref[...]

    N, D = q.shape
    nb = N // _BS
    scale = D ** (-0.5)
    f32 = jnp.float32
    bf16 = jnp.bfloat16

    kc = jnp.mean(k.reshape(nb, _BS, D), axis=1)
    kc0 = jnp.mean(k0.reshape(nb, _BS, D), axis=1)
    vc0 = jnp.mean(v0.reshape(nb, _BS, D), axis=1)

    q16 = q.astype(bf16)
    k16 = k.astype(bf16)
    v16 = v.astype(bf16)

    gates = jax.nn.sigmoid(jnp.dot(q16, gw.astype(bf16),
                                   preferred_element_type=f32))
    g_cmp = gates[:, 0:1]
    g_slc = gates[:, 1:2]

    s_sel_t = _dot_nt(kc.astype(bf16), q16) * scale
    j_sub = jax.lax.broadcasted_iota(jnp.int32, (nb, N), 0)
    m_lane = jax.lax.broadcasted_iota(jnp.int32, (nb, N), 1)
    blk_causal_t = (m_lane // _BS) >= j_sub
    s_m = jnp.where(blk_causal_t, s_sel_t, _NEG)
    sel_t = jnp.zeros((nb, N), dtype=jnp.bool_)
    for _ in range(_S):
        smax = jnp.max(s_m, axis=0, keepdims=True)
        is_max = s_m == smax
        first = jnp.min(jnp.where(is_max, j_sub, nb), axis=0, keepdims=True)
        pick = j_sub == first
        valid = smax > (_NEG * 0.5)
        sel_t = jnp.logical_or(sel_t, jnp.logical_and(pick, valid))
        s_m = jnp.where(pick, _NEG, s_m)

    sc0 = _dot_nt(q16, kc0.astype(bf16)) * scale
    qb = jax.lax.broadcasted_iota(jnp.int32, (N, nb), 0) // _BS
    jb = jax.lax.broadcasted_iota(jnp.int32, (N, nb), 1)
    p_cmp = jnp.where(qb >= jb, _silu(sc0), 0.0)
    o_cmp = jnp.dot(p_cmp.astype(bf16), vc0.astype(bf16),
                    preferred_element_type=f32)

    s_tok_t = (_dot_nt(k16, q16) * scale).astype(bf16)
    blk_of_t = jax.lax.broadcasted_iota(jnp.int32, (N, nb), 0) // _BS
    j_col = jax.lax.broadcasted_iota(jnp.int32, (N, nb), 1)
    ind_tok = (blk_of_t == j_col).astype(bf16)
    sel_exp_t = jnp.dot(ind_tok, sel_t.astype(bf16),
                        preferred_element_type=f32).astype(bf16)
    p_t = _silu(s_tok_t) * (sel_exp_t * causal_t)
    o_slc = _dot_tt(p_t, v16)

    o_ref[0, 0] = o_cmp * g_cmp + o_slc * g_slc


def kernel(jagged_q, jagged_k, jagged_v, jagged_u, padded_q, padded_k,
           padded_v, x_offsets, gate_w, padding_mask):
    B, N, H, D = padded_q.shape
    qt = padded_q.transpose(0, 2, 1, 3)
    kt = padded_k.transpose(0, 2, 1, 3)
    vt = padded_v.transpose(0, 2, 1, 3)
    causal_t = (jnp.arange(N)[None, :] >= jnp.arange(N)[:, None]
                ).astype(jnp.bfloat16)

    bhspec = pl.BlockSpec((1, 1, N, D), lambda h, b: (b, h, 0, 0))
    b0spec = pl.BlockSpec((1, 1, N, D), lambda h, b: (0, h, 0, 0))
    gwspec = pl.BlockSpec((1, D, 3), lambda h, b: (h, 0, 0))
    cspec = pl.BlockSpec((N, N), lambda h, b: (0, 0))

    out = pl.pallas_call(
        _fwd,
        grid=(H, B),
        in_specs=[bhspec, bhspec, bhspec, b0spec, b0spec, gwspec, cspec],
        out_specs=bhspec,
        out_shape=jax.ShapeDtypeStruct((B, H, N, D), jnp.float32),
        compiler_params=pltpu.CompilerParams(
            dimension_semantics=("parallel", "parallel")),
    )(qt, kt, vt, kt, vt, gate_w, causal_t)

    return out.transpose(0, 2, 1, 3).reshape(B * N, H, D)
